# baseline (speedup 1.0000x reference)
import functools

import jax
import jax.numpy as jnp
from jax import lax
from jax.experimental import pallas as pl
from jax.experimental.pallas import tpu as pltpu
from jax.experimental.pallas import tpu_sc as plsc

_B, _S, _V = 64, 50, 10000
_N = _B * _S
_NC, _NS, _L = 2, 16, 16
_NW = _NC * _NS
_PW = 112
_P = _NW * _PW


def _sc_partials(x_flat, tgt_pad, msk_pad):
    mesh = plsc.VectorSubcoreMesh(core_axis_name="c", subcore_axis_name="s")

    @functools.partial(
        pl.kernel,
        mesh=mesh,
        out_type=jax.ShapeDtypeStruct((2, _NW, _L), jnp.float32),
        scratch_types=[
            pltpu.VMEM((_PW,), jnp.int32),
            pltpu.VMEM((_PW,), jnp.int32),
            pltpu.VMEM((_PW,), jnp.float32),
            pltpu.VMEM((_PW,), jnp.float32),
            pltpu.VMEM((_L,), jnp.float32),
            pltpu.VMEM((_L,), jnp.float32),
        ],
    )
    def k(x_hbm, t_hbm, m_hbm, out_hbm, idx_v, tgt_v, msk_v, val_v, accv, maccv):
        wid = lax.axis_index("s") * _NC + lax.axis_index("c")
        base = wid * _PW
        pltpu.sync_copy(t_hbm.at[pl.ds(base, _PW)], tgt_v)
        pltpu.sync_copy(m_hbm.at[pl.ds(base, _PW)], msk_v)
        lanes = lax.iota(jnp.int32, _L)
        for j in range(_PW // _L):
            g = base + j * _L + lanes
            fi = g * _V + tgt_v[pl.ds(j * _L, _L)]
            idx_v[pl.ds(j * _L, _L)] = jnp.minimum(fi, _N * _V - 1)
        pltpu.sync_copy(x_hbm.at[idx_v], val_v)
        acc = val_v[pl.ds(0, _L)] * msk_v[pl.ds(0, _L)]
        macc = msk_v[pl.ds(0, _L)]
        for j in range(1, _PW // _L):
            acc = acc + val_v[pl.ds(j * _L, _L)] * msk_v[pl.ds(j * _L, _L)]
            macc = macc + msk_v[pl.ds(j * _L, _L)]
        accv[...] = acc
        maccv[...] = macc
        pltpu.sync_copy(accv, out_hbm.at[0, wid])
        pltpu.sync_copy(maccv, out_hbm.at[1, wid])

    return k(x_flat, tgt_pad, msk_pad)


def _tc_finish(parts):
    def body(p_ref, o_ref):
        sv = jnp.sum(p_ref[0])
        sm = jnp.sum(p_ref[1])
        o_ref[0] = -sv / sm

    return pl.pallas_call(
        body,
        out_shape=jax.ShapeDtypeStruct((1,), jnp.float32),
        out_specs=pl.BlockSpec(memory_space=pltpu.SMEM),
    )(parts)


def kernel(_input, frame_weight, att_weight, frame_surpervise, att_surpervise,
           target, mask, ada_frame_out, ada_att_out, pre_probs, probs):
    tgt = target[:, :_input.shape[1]].astype(jnp.int32)
    m = mask[:, :_input.shape[1]].astype(_input.dtype)
    x_flat = _input.reshape(-1)
    tgt_pad = jnp.pad(tgt.reshape(-1), (0, _P - _N))
    msk_pad = jnp.pad(m.reshape(-1), (0, _P - _N))
    parts = _sc_partials(x_flat, tgt_pad, msk_pad)
    out = _tc_finish(parts)[0]
    return (out, out, out, out)
